# baseline (speedup 1.0000x reference)
import jax
import jax.numpy as jnp
from jax import lax
from jax.experimental import pallas as pl
from jax.experimental.pallas import tpu as pltpu
from jax.experimental.pallas import tpu_sc as plsc

N_DIMS = 64
NB_Q = 100000
NB_S = 1000
NB_R = 2
SEQ_LEN = 200
BATCH = 1024

TOKENS = BATCH * SEQ_LEN
NC = 2
NS = 16
NW = NC * NS
TOK_W = TOKENS // NW
NBLK = 400
NBLOCKS = TOK_W // NBLK
NCH = N_DIMS // 16
GRP = 16


def _sc_body(e_in, s_in, rf_in, t_in, e_out, s_out,
             exe_tab, skill_tab, resp_tab, w_row, b_vec, pos_tab, zeros_hbm,
             enc_o, dec_o, out_o,
             eidx_v, sidx_v, eidx2_v, sidx2_v, rf_v, t_v,
             bufA, bufB, bufC, pos_v, resp_v, w_v, b_v,
             semA, semB, semP, semS):
    wid = lax.axis_index("s") * NC + lax.axis_index("c")
    wbase = wid * TOK_W

    pltpu.sync_copy(pos_tab, pos_v)
    pltpu.sync_copy(resp_tab, resp_v)
    pltpu.sync_copy(w_row, w_v)
    pltpu.sync_copy(b_vec, b_v)
    pltpu.sync_copy(e_in.at[pl.ds(wbase, TOK_W)], eidx_v)
    pltpu.sync_copy(s_in.at[pl.ds(wbase, TOK_W)], sidx_v)
    pltpu.sync_copy(e_out.at[pl.ds(wbase, TOK_W)], eidx2_v)
    pltpu.sync_copy(s_out.at[pl.ds(wbase, TOK_W)], sidx2_v)
    pltpu.sync_copy(rf_in.at[pl.ds(wbase, TOK_W)], rf_v)
    pltpu.sync_copy(t_in.at[pl.ds(wbase, TOK_W)], t_v)

    WV = [w_v[0, pl.ds(c * 16, 16)] for c in range(NCH)]
    R0 = [resp_v[0, pl.ds(c * 16, 16)] + b_v[pl.ds(c * 16, 16)]
          for c in range(NCH)]
    DF = [resp_v[1, pl.ds(c * 16, 16)] - resp_v[0, pl.ds(c * 16, 16)]
          for c in range(NCH)]

    def block(blk, carry):
        base = wbase + blk * NBLK
        bsl = pl.ds(blk * NBLK, NBLK)

        @pl.when(blk > 0)
        def _():
            pltpu.make_async_copy(bufA, enc_o.at[pl.ds(base - NBLK, NBLK)],
                                  semS).wait()
            pltpu.make_async_copy(bufB, out_o.at[pl.ds(base - NBLK, NBLK)],
                                  semS).wait()
            pltpu.make_async_copy(bufC, dec_o.at[pl.ds(base - NBLK, NBLK)],
                                  semS).wait()

        dA1 = pltpu.async_copy(pos_tab, bufA.at[pl.ds(0, SEQ_LEN)], semP)
        dA2 = pltpu.async_copy(pos_tab, bufA.at[pl.ds(SEQ_LEN, SEQ_LEN)], semP)
        dB = pltpu.async_copy(zeros_hbm, bufB, semP)

        dA1.wait()
        dA2.wait()
        gA1 = pltpu.async_copy(exe_tab.at[eidx_v.at[bsl]], bufA, semA,
                               add=True)
        gA2 = pltpu.async_copy(skill_tab.at[sidx_v.at[bsl]], bufA, semA,
                               add=True)
        dB.wait()
        gB1 = pltpu.async_copy(exe_tab.at[eidx2_v.at[bsl]], bufB, semB,
                               add=True)
        gB2 = pltpu.async_copy(skill_tab.at[sidx2_v.at[bsl]], bufB, semB,
                               add=True)

        def dec_grp(g, _):
            i0 = g * GRP
            rvec = rf_v[pl.ds(blk * NBLK + i0, GRP)]
            tvec = t_v[pl.ds(blk * NBLK + i0, GRP)]
            for j in range(GRP):
                i = i0 + j
                l = lax.rem(blk * NBLK + i, SEQ_LEN)
                r_i = rvec[j]
                t_i = tvec[j]
                for c in range(NCH):
                    sl = pl.ds(c * 16, 16)
                    bufC[i, sl] = (pos_v[l, sl] + R0[c]
                                   + r_i * DF[c] + t_i * WV[c])
            return 0
        lax.fori_loop(0, NBLK // GRP, dec_grp, 0)

        gA1.wait()
        gA2.wait()
        pltpu.async_copy(bufA, enc_o.at[pl.ds(base, NBLK)], semS)
        gB1.wait()
        gB2.wait()
        pltpu.async_copy(bufB, out_o.at[pl.ds(base, NBLK)], semS)
        pltpu.async_copy(bufC, dec_o.at[pl.ds(base, NBLK)], semS)
        return carry

    lax.fori_loop(0, NBLOCKS, block, 0)

    last = wbase + (NBLOCKS - 1) * NBLK
    pltpu.make_async_copy(bufA, enc_o.at[pl.ds(last, NBLK)], semS).wait()
    pltpu.make_async_copy(bufB, out_o.at[pl.ds(last, NBLK)], semS).wait()
    pltpu.make_async_copy(bufC, dec_o.at[pl.ds(last, NBLK)], semS).wait()


@jax.jit
def _run(e_in, s_in, rf_in, t_in, e_out, s_out,
         exe_tab, skill_tab, resp_tab, w_row, b_vec, pos_tab, zeros_hbm):
    f32 = jnp.float32
    mesh = plsc.VectorSubcoreMesh(core_axis_name="c", subcore_axis_name="s",
                                  num_cores=NC, num_subcores=NS)
    out_type = (jax.ShapeDtypeStruct((TOKENS, N_DIMS), f32),
                jax.ShapeDtypeStruct((TOKENS, N_DIMS), f32),
                jax.ShapeDtypeStruct((TOKENS, N_DIMS), f32))
    scratch = [
        pltpu.VMEM((TOK_W,), jnp.int32),
        pltpu.VMEM((TOK_W,), jnp.int32),
        pltpu.VMEM((TOK_W,), jnp.int32),
        pltpu.VMEM((TOK_W,), jnp.int32),
        pltpu.VMEM((TOK_W,), f32),
        pltpu.VMEM((TOK_W,), f32),
        pltpu.VMEM((NBLK, N_DIMS), f32),
        pltpu.VMEM((NBLK, N_DIMS), f32),
        pltpu.VMEM((NBLK, N_DIMS), f32),
        pltpu.VMEM((SEQ_LEN, N_DIMS), f32),
        pltpu.VMEM((NB_R, N_DIMS), f32),
        pltpu.VMEM((1, N_DIMS), f32),
        pltpu.VMEM((N_DIMS,), f32),
        pltpu.SemaphoreType.DMA,
        pltpu.SemaphoreType.DMA,
        pltpu.SemaphoreType.DMA,
        pltpu.SemaphoreType.DMA,
    ]
    run = pl.kernel(_sc_body, out_type=out_type, mesh=mesh,
                    scratch_types=scratch,
                    compiler_params=pltpu.CompilerParams(
                        use_tc_tiling_on_sc=False))
    return run(e_in, s_in, rf_in, t_in, e_out, s_out,
               exe_tab, skill_tab, resp_tab, w_row, b_vec, pos_tab,
               zeros_hbm)


def kernel(input_exercise, input_skill, input_r, in_elapsed_time,
           out_exercise, out_skill, exercise_table, skill_table,
           response_table, elapsed_W, elapsed_b, position_table):
    e_in = input_exercise.reshape(TOKENS)
    s_in = input_skill.reshape(TOKENS)
    rf_in = input_r.reshape(TOKENS).astype(jnp.float32)
    t_in = in_elapsed_time.reshape(TOKENS)
    e_out = out_exercise.reshape(TOKENS)
    s_out = out_skill.reshape(TOKENS)
    zeros_hbm = jnp.zeros((NBLK, N_DIMS), jnp.float32)

    enc, dec, out = _run(e_in, s_in, rf_in, t_in, e_out, s_out,
                         exercise_table, skill_table, response_table,
                         elapsed_W, elapsed_b, position_table, zeros_hbm)
    shp = (BATCH, SEQ_LEN, N_DIMS)
    return (enc.reshape(shp), dec.reshape(shp), out.reshape(shp))


# speedup vs baseline: 1.0283x; 1.0283x over previous
import jax
import jax.numpy as jnp
from jax import lax
from jax.experimental import pallas as pl
from jax.experimental.pallas import tpu as pltpu
from jax.experimental.pallas import tpu_sc as plsc

N_DIMS = 64
NB_Q = 100000
NB_S = 1000
NB_R = 2
SEQ_LEN = 200
BATCH = 1024

TOKENS = BATCH * SEQ_LEN
NC = 2
NS = 16
NW = NC * NS
TOK_W = TOKENS // NW
NBLK = SEQ_LEN
NBLOCKS = TOK_W // NBLK
NPAIR = NBLOCKS // 2
NCH = N_DIMS // 16
GRP = 16


def _sc_body(e_in, s_in, rf_in, t_in, e_out, s_out,
             exe_tab, skill_tab, resp_tab, w_row, b_vec, pos_tab,
             enc_o, dec_o, out_o,
             eidx_v, sidx_v, eidx2_v, sidx2_v, rf_v, t_v,
             bufA, bufB, bufC, pos_v, resp_v, w_v, b_v,
             semA, semB, semSAB, semSC):
    wid = lax.axis_index("s") * NC + lax.axis_index("c")
    wbase = wid * TOK_W

    pltpu.sync_copy(pos_tab, pos_v)
    pltpu.sync_copy(resp_tab, resp_v)
    pltpu.sync_copy(w_row, w_v)
    pltpu.sync_copy(b_vec, b_v)
    pltpu.sync_copy(e_in.at[pl.ds(wbase, TOK_W)], eidx_v)
    pltpu.sync_copy(s_in.at[pl.ds(wbase, TOK_W)], sidx_v)
    pltpu.sync_copy(e_out.at[pl.ds(wbase, TOK_W)], eidx2_v)
    pltpu.sync_copy(s_out.at[pl.ds(wbase, TOK_W)], sidx2_v)
    pltpu.sync_copy(rf_in.at[pl.ds(wbase, TOK_W)], rf_v)
    pltpu.sync_copy(t_in.at[pl.ds(wbase, TOK_W)], t_v)

    zv = jnp.zeros((16,), jnp.float32)
    WV = [w_v[0, pl.ds(c * 16, 16)] for c in range(NCH)]
    R0 = [resp_v[0, pl.ds(c * 16, 16)] + b_v[pl.ds(c * 16, 16)]
          for c in range(NCH)]
    DF = [resp_v[1, pl.ds(c * 16, 16)] - resp_v[0, pl.ds(c * 16, 16)]
          for c in range(NCH)]

    A = [bufA.at[0], bufA.at[1]]
    B = [bufB.at[0], bufB.at[1]]
    C = [bufC.at[0], bufC.at[1]]

    def prefill(p):
        def fill_row(i, _):
            for c in range(NCH):
                sl = pl.ds(c * 16, 16)
                A[p][i, sl] = pos_v[i, sl]
                B[p][i, sl] = zv
            return 0
        lax.fori_loop(0, NBLK, fill_row, 0, unroll=2)

    def issue_gathers(p, blk):
        bsl = pl.ds(blk * NBLK, NBLK)
        pltpu.async_copy(exe_tab.at[eidx_v.at[bsl]], A[p], semA, add=True)
        pltpu.async_copy(skill_tab.at[sidx_v.at[bsl]], A[p], semA, add=True)
        pltpu.async_copy(exe_tab.at[eidx2_v.at[bsl]], B[p], semB, add=True)
        pltpu.async_copy(skill_tab.at[sidx2_v.at[bsl]], B[p], semB, add=True)

    def wait_gathers(p, blk):
        bsl = pl.ds(blk * NBLK, NBLK)
        pltpu.make_async_copy(exe_tab.at[eidx_v.at[bsl]], A[p], semA).wait()
        pltpu.make_async_copy(skill_tab.at[sidx_v.at[bsl]], A[p], semA).wait()
        pltpu.make_async_copy(exe_tab.at[eidx2_v.at[bsl]], B[p], semB).wait()
        pltpu.make_async_copy(skill_tab.at[sidx2_v.at[bsl]], B[p], semB).wait()

    def issue_scatters(p, base):
        osl = pl.ds(base, NBLK)
        pltpu.async_copy(A[p], enc_o.at[osl], semSAB)
        pltpu.async_copy(B[p], out_o.at[osl], semSAB)
        pltpu.async_copy(C[p], dec_o.at[osl], semSC)

    def wait_scatters_ab(p, base):
        osl = pl.ds(base, NBLK)
        pltpu.make_async_copy(A[p], enc_o.at[osl], semSAB).wait()
        pltpu.make_async_copy(B[p], out_o.at[osl], semSAB).wait()

    def wait_scatter_c(p, base):
        osl = pl.ds(base, NBLK)
        pltpu.make_async_copy(C[p], dec_o.at[osl], semSC).wait()

    def dec_pass(p, blk):
        def dec_grp(g, _):
            i0 = g * GRP
            rvec = rf_v[pl.ds(blk * NBLK + i0, GRP)]
            tvec = t_v[pl.ds(blk * NBLK + i0, GRP)]
            for j in range(GRP):
                i = i0 + j
                r_i = rvec[j]
                t_i = tvec[j]
                for c in range(NCH):
                    sl = pl.ds(c * 16, 16)
                    C[p][i, sl] = (pos_v[i, sl] + R0[c]
                                   + r_i * DF[c] + t_i * WV[c])
            return 0
        lax.fori_loop(0, NBLK // GRP, dec_grp, 0)

    prefill(0)
    issue_gathers(0, 0)

    def step(k, p):
        base = wbase + k * NBLK
        @pl.when(k >= 2)
        def _():
            wait_scatter_c(p, base - 2 * NBLK)
        dec_pass(p, k)
        @pl.when(k >= 1)
        def _():
            wait_scatters_ab(1 - p, base - NBLK)
        @pl.when(k + 1 < NBLOCKS)
        def _():
            prefill(1 - p)
            issue_gathers(1 - p, k + 1)
        wait_gathers(p, k)
        issue_scatters(p, base)

    def pair(g, carry):
        step(2 * g, 0)
        step(2 * g + 1, 1)
        return carry

    lax.fori_loop(0, NPAIR, pair, 0)

    wait_scatter_c(0, wbase + (NBLOCKS - 2) * NBLK)
    wait_scatters_ab(1, wbase + (NBLOCKS - 1) * NBLK)
    wait_scatter_c(1, wbase + (NBLOCKS - 1) * NBLK)


@jax.jit
def _run(e_in, s_in, rf_in, t_in, e_out, s_out,
         exe_tab, skill_tab, resp_tab, w_row, b_vec, pos_tab):
    f32 = jnp.float32
    mesh = plsc.VectorSubcoreMesh(core_axis_name="c", subcore_axis_name="s",
                                  num_cores=NC, num_subcores=NS)
    out_type = (jax.ShapeDtypeStruct((TOKENS, N_DIMS), f32),
                jax.ShapeDtypeStruct((TOKENS, N_DIMS), f32),
                jax.ShapeDtypeStruct((TOKENS, N_DIMS), f32))
    scratch = [
        pltpu.VMEM((TOK_W,), jnp.int32),
        pltpu.VMEM((TOK_W,), jnp.int32),
        pltpu.VMEM((TOK_W,), jnp.int32),
        pltpu.VMEM((TOK_W,), jnp.int32),
        pltpu.VMEM((TOK_W,), f32),
        pltpu.VMEM((TOK_W,), f32),
        pltpu.VMEM((2, NBLK, N_DIMS), f32),
        pltpu.VMEM((2, NBLK, N_DIMS), f32),
        pltpu.VMEM((2, NBLK, N_DIMS), f32),
        pltpu.VMEM((SEQ_LEN, N_DIMS), f32),
        pltpu.VMEM((NB_R, N_DIMS), f32),
        pltpu.VMEM((1, N_DIMS), f32),
        pltpu.VMEM((N_DIMS,), f32),
        pltpu.SemaphoreType.DMA,
        pltpu.SemaphoreType.DMA,
        pltpu.SemaphoreType.DMA,
        pltpu.SemaphoreType.DMA,
    ]
    run = pl.kernel(_sc_body, out_type=out_type, mesh=mesh,
                    scratch_types=scratch,
                    compiler_params=pltpu.CompilerParams(
                        use_tc_tiling_on_sc=False))
    return run(e_in, s_in, rf_in, t_in, e_out, s_out,
               exe_tab, skill_tab, resp_tab, w_row, b_vec, pos_tab)


def kernel(input_exercise, input_skill, input_r, in_elapsed_time,
           out_exercise, out_skill, exercise_table, skill_table,
           response_table, elapsed_W, elapsed_b, position_table):
    e_in = input_exercise.reshape(TOKENS)
    s_in = input_skill.reshape(TOKENS)
    rf_in = input_r.reshape(TOKENS).astype(jnp.float32)
    t_in = in_elapsed_time.reshape(TOKENS)
    e_out = out_exercise.reshape(TOKENS)
    s_out = out_skill.reshape(TOKENS)

    enc, dec, out = _run(e_in, s_in, rf_in, t_in, e_out, s_out,
                         exercise_table, skill_table, response_table,
                         elapsed_W, elapsed_b, position_table)
    shp = (BATCH, SEQ_LEN, N_DIMS)
    return (enc.reshape(shp), dec.reshape(shp), out.reshape(shp))
